# baseline (speedup 1.0000x reference)
import functools

import jax
import jax.numpy as jnp
from jax import lax
from jax.experimental import pallas as pl
from jax.experimental.pallas import tpu as pltpu
from jax.experimental.pallas import tpu_sc as plsc

N = 10000
NPAD = 10240
E = 320000
K = 8
D = 16
TILES = 16
RPT = NPAD // TILES
CHUNK = 128
SCH = 158
RCH = SCH + 2
EPT = SCH * CHUNK
EPAD = EPT * TILES


def _rsqrt16(x):
    i = plsc.bitcast(x, jnp.int32)
    i = 0x5F3759DF - lax.shift_right_logical(i, 1)
    y = plsc.bitcast(i, jnp.float32)
    for _ in range(3):
        y = y * (1.5 - 0.5 * x * y * y)
    return y


def _sc_propagate(z, row_g, col_g, bias16):
    mesh = plsc.VectorSubcoreMesh(
        core_axis_name="c", subcore_axis_name="s", num_cores=1
    )

    @functools.partial(
        pl.kernel,
        out_type=jax.ShapeDtypeStruct((NPAD, D), jnp.float32),
        mesh=mesh,
        scratch_types=[
            pltpu.VMEM_SHARED((NPAD, D), jnp.float32),
            pltpu.VMEM_SHARED((NPAD, D), jnp.float32),
            pltpu.VMEM((RCH, CHUNK), jnp.int32),
            pltpu.VMEM((SCH, CHUNK), jnp.int32),
            pltpu.VMEM((CHUNK, D), jnp.float32),
            pltpu.VMEM((CHUNK, D), jnp.float32),
            pltpu.VMEM((RPT, D), jnp.float32),
            pltpu.VMEM((RPT, D), jnp.float32),
            pltpu.VMEM((RPT, D), jnp.float32),
            pltpu.VMEM((RPT, D), jnp.float32),
            pltpu.VMEM((D,), jnp.float32),
            pltpu.SemaphoreType.DMA,
            pltpu.SemaphoreType.DMA,
            pltpu.SemaphoreType.DMA,
            pltpu.SemaphoreType.DMA,
        ],
        compiler_params=pltpu.CompilerParams(
            needs_layout_passes=False, use_tc_tiling_on_sc=False
        ),
    )
    def body(z_hbm, row_hbm, col_hbm, bias_hbm, h_hbm,
             nxt_sh, t_sh, row_v, col_v, msg0, msg1, tv, hv, dv, av, bv,
             sem0, sem1, ssem0, ssem1):
        tid = lax.axis_index("s")
        sl = pl.ds(tid * RPT, RPT)

        pltpu.sync_copy(row_hbm.at[tid], row_v)
        pltpu.sync_copy(col_hbm.at[tid], col_v)
        pltpu.sync_copy(bias_hbm, bv)

        def fill_ones(i, c):
            msg0[i, :] = jnp.ones((D,), jnp.float32)
            return c

        def fill_zero(i, c):
            av[i, :] = jnp.zeros((D,), jnp.float32)
            return c

        lax.fori_loop(0, CHUNK, fill_ones, 0)
        lax.fori_loop(0, RPT, fill_zero, 0)
        pltpu.sync_copy(av, nxt_sh.at[sl])
        plsc.subcore_barrier()

        def deg_fire(j, c):
            pltpu.async_copy(msg0, nxt_sh.at[col_v.at[j]], ssem_d, add=True)
            return c

        def deg_drain(j, c):
            pltpu.make_async_copy(msg0, nxt_sh.at[pl.ds(0, CHUNK)],
                                  ssem_d).wait()
            return c

        ssem_d = sem1
        lax.fori_loop(0, SCH, deg_fire, 0)
        lax.fori_loop(0, SCH, deg_drain, 0)
        plsc.subcore_barrier()

        pltpu.sync_copy(nxt_sh.at[sl], av)
        pltpu.sync_copy(z_hbm.at[sl], tv)
        b = bv[:]

        def init_rows(i, c):
            deg = av[i, :] + 1.0
            y = _rsqrt16(deg)
            dv[i, :] = y
            tv[i, :] = y * tv[i, :]
            hv[i, :] = b
            av[i, :] = jnp.zeros((D,), jnp.float32)
            return c

        lax.fori_loop(0, RPT, init_rows, 0)
        pltpu.sync_copy(tv, t_sh.at[sl])
        pltpu.sync_copy(av, nxt_sh.at[sl])
        plsc.subcore_barrier()

        def gather(j, buf, sem):
            pltpu.async_copy(t_sh.at[row_v.at[j]], buf, sem)

        def gwait(buf, sem):
            pltpu.make_async_copy(t_sh.at[pl.ds(0, CHUNK)], buf, sem).wait()

        def scatter(j, buf, sem):
            pltpu.async_copy(buf, nxt_sh.at[col_v.at[j]], sem, add=True)

        def swait(buf, sem):
            pltpu.make_async_copy(buf, nxt_sh.at[pl.ds(0, CHUNK)], sem).wait()

        def hop(k, c):
            gather(0, msg0, sem0)
            gather(1, msg1, sem1)

            def pipe(i, cc):
                jj = 2 * i
                gwait(msg0, sem0)
                scatter(jj, msg0, ssem0)
                gwait(msg1, sem1)
                scatter(jj + 1, msg1, ssem1)
                swait(msg0, ssem0)
                gather(jj + 2, msg0, sem0)
                swait(msg1, ssem1)
                gather(jj + 3, msg1, sem1)
                return cc

            lax.fori_loop(0, SCH // 2, pipe, 0)
            gwait(msg0, sem0)
            gwait(msg1, sem1)
            plsc.subcore_barrier()

            pltpu.sync_copy(nxt_sh.at[sl], av)

            def rowu(i, cc):
                cur = dv[i, :] * (av[i, :] + tv[i, :])
                hv[i, :] = hv[i, :] + cur
                tv[i, :] = dv[i, :] * cur
                av[i, :] = jnp.zeros((D,), jnp.float32)
                return cc

            lax.fori_loop(0, RPT, rowu, 0)
            pltpu.sync_copy(tv, t_sh.at[sl])
            pltpu.sync_copy(av, nxt_sh.at[sl])
            plsc.subcore_barrier()
            return c

        lax.fori_loop(0, K, hop, 0)
        pltpu.sync_copy(hv, h_hbm.at[sl])

    return body(z, row_g, col_g, bias16)


def _tc_project(x_pad, wt):
    def mm(x_ref, w_ref, o_ref):
        o_ref[:, :] = jnp.dot(x_ref[:, :], w_ref[:, :],
                              preferred_element_type=jnp.float32)

    blk = 1024
    return pl.pallas_call(
        mm,
        grid=(NPAD // blk,),
        in_specs=[
            pl.BlockSpec((blk, 128), lambda i: (i, 0)),
            pl.BlockSpec((128, D), lambda i: (0, 0)),
        ],
        out_specs=pl.BlockSpec((blk, D), lambda i: (i, 0)),
        out_shape=jax.ShapeDtypeStruct((NPAD, D), jnp.float32),
    )(x_pad, wt)


def kernel(x, edge_index, conv_w, conv_b, lin_w, lin_b):
    f32 = jnp.float32
    wc = (lin_w @ conv_w) * (1.0 / K)
    bias = lin_w @ conv_b + lin_b
    bias16 = jnp.zeros((D,), f32).at[:7].set(bias)
    wcp = jnp.zeros((D, 128), f32).at[:7, :].set(wc)

    x_pad = jnp.zeros((NPAD, 128), f32).at[:N].set(x)
    z = _tc_project(x_pad, wcp.T)

    row = edge_index[0]
    col = edge_index[1]
    pad = jnp.full((EPAD - E,), N, jnp.int32)
    row_p = jnp.concatenate([row, pad]).reshape(TILES, SCH, CHUNK)
    row_g = jnp.zeros((TILES, RCH, CHUNK), jnp.int32).at[:, :SCH, :].set(row_p)
    col_g = jnp.concatenate([col, pad]).reshape(TILES, SCH, CHUNK)

    h = _sc_propagate(z, row_g, col_g, bias16)
    return h[:N, :7]


# speedup vs baseline: 1.0711x; 1.0711x over previous
import functools

import jax
import jax.numpy as jnp
from jax import lax
from jax.experimental import pallas as pl
from jax.experimental.pallas import tpu as pltpu
from jax.experimental.pallas import tpu_sc as plsc

N = 10000
NPAD = 10240
E = 320000
K = 8
D = 16
TILES = 16
RPT = NPAD // TILES
CHUNK = 128
SCH = 160
RCH = SCH + 4
EPT = SCH * CHUNK
EPAD = EPT * TILES


def _rsqrt16(x):
    i = plsc.bitcast(x, jnp.int32)
    i = 0x5F3759DF - lax.shift_right_logical(i, 1)
    y = plsc.bitcast(i, jnp.float32)
    for _ in range(3):
        y = y * (1.5 - 0.5 * x * y * y)
    return y


def _sc_propagate(z, row_g, col_g, bias16):
    mesh = plsc.VectorSubcoreMesh(
        core_axis_name="c", subcore_axis_name="s", num_cores=1
    )

    @functools.partial(
        pl.kernel,
        out_type=jax.ShapeDtypeStruct((NPAD, D), jnp.float32),
        mesh=mesh,
        scratch_types=[
            pltpu.VMEM_SHARED((NPAD, D), jnp.float32),
            pltpu.VMEM_SHARED((NPAD, D), jnp.float32),
            pltpu.VMEM((RCH, CHUNK), jnp.int32),
            pltpu.VMEM((SCH, CHUNK), jnp.int32),
            [pltpu.VMEM((CHUNK, D), jnp.float32)] * 4,
            pltpu.VMEM((RPT, D), jnp.float32),
            pltpu.VMEM((RPT, D), jnp.float32),
            pltpu.VMEM((RPT, D), jnp.float32),
            pltpu.VMEM((RPT, D), jnp.float32),
            pltpu.VMEM((D,), jnp.float32),
            [pltpu.SemaphoreType.DMA] * 4,
        ],
        compiler_params=pltpu.CompilerParams(
            needs_layout_passes=False, use_tc_tiling_on_sc=False
        ),
    )
    def body(z_hbm, row_hbm, col_hbm, bias_hbm, h_hbm,
             nxt_sh, t_sh, row_v, col_v, msg, tv, hv, dv, av, bv, sem):
        tid = lax.axis_index("s")
        sl = pl.ds(tid * RPT, RPT)

        pltpu.sync_copy(row_hbm.at[tid], row_v)
        pltpu.sync_copy(col_hbm.at[tid], col_v)
        pltpu.sync_copy(bias_hbm, bv)

        def fill_ones(i, c):
            msg[0][i, :] = jnp.ones((D,), jnp.float32)
            return c

        def fill_zero(i, c):
            av[i, :] = jnp.zeros((D,), jnp.float32)
            return c

        lax.fori_loop(0, CHUNK, fill_ones, 0)
        lax.fori_loop(0, RPT, fill_zero, 0)
        pltpu.sync_copy(av, nxt_sh.at[sl])
        plsc.subcore_barrier()

        def deg_fire(j, c):
            pltpu.async_copy(msg[0], nxt_sh.at[col_v.at[j]], ssem_d, add=True)
            return c

        def deg_drain(j, c):
            pltpu.make_async_copy(msg[0], nxt_sh.at[pl.ds(0, CHUNK)],
                                  ssem_d).wait()
            return c

        ssem_d = sem[1]
        lax.fori_loop(0, SCH, deg_fire, 0)
        lax.fori_loop(0, SCH, deg_drain, 0)
        plsc.subcore_barrier()

        pltpu.sync_copy(nxt_sh.at[sl], av)
        pltpu.sync_copy(z_hbm.at[sl], tv)
        b = bv[:]

        def init_rows(i, c):
            deg = av[i, :] + 1.0
            y = _rsqrt16(deg)
            dv[i, :] = y
            tv[i, :] = y * tv[i, :]
            hv[i, :] = b
            av[i, :] = jnp.zeros((D,), jnp.float32)
            return c

        lax.fori_loop(0, RPT, init_rows, 0)
        pltpu.sync_copy(tv, t_sh.at[sl])
        pltpu.sync_copy(av, nxt_sh.at[sl])
        plsc.subcore_barrier()

        def gather(j, buf, sem):
            pltpu.async_copy(t_sh.at[row_v.at[j]], buf, sem)

        def gwait(buf, sem):
            pltpu.make_async_copy(t_sh.at[pl.ds(0, CHUNK)], buf, sem).wait()

        def hop(k, c):
            for b_ in range(4):
                gather(b_, msg[b_], sem[b_])

            def pipe(i, cc):
                jj = 4 * i
                for b_ in range(4):
                    gwait(msg[b_], sem[b_])
                    pltpu.sync_copy(msg[b_], nxt_sh.at[col_v.at[jj + b_]],
                                    add=True)
                    gather(jj + 4 + b_, msg[b_], sem[b_])
                return cc

            lax.fori_loop(0, SCH // 4, pipe, 0)
            for b_ in range(4):
                gwait(msg[b_], sem[b_])
            plsc.subcore_barrier()

            pltpu.sync_copy(nxt_sh.at[sl], av)

            def rowu(i, cc):
                cur = dv[i, :] * (av[i, :] + tv[i, :])
                hv[i, :] = hv[i, :] + cur
                tv[i, :] = dv[i, :] * cur
                av[i, :] = jnp.zeros((D,), jnp.float32)
                return cc

            lax.fori_loop(0, RPT, rowu, 0)
            pltpu.sync_copy(tv, t_sh.at[sl])
            pltpu.sync_copy(av, nxt_sh.at[sl])
            plsc.subcore_barrier()
            return c

        lax.fori_loop(0, K, hop, 0)
        pltpu.sync_copy(hv, h_hbm.at[sl])

    return body(z, row_g, col_g, bias16)


def _tc_project(x_pad, wt):
    def mm(x_ref, w_ref, o_ref):
        o_ref[:, :] = jnp.dot(x_ref[:, :], w_ref[:, :],
                              preferred_element_type=jnp.float32)

    blk = 1024
    return pl.pallas_call(
        mm,
        grid=(NPAD // blk,),
        in_specs=[
            pl.BlockSpec((blk, 128), lambda i: (i, 0)),
            pl.BlockSpec((128, D), lambda i: (0, 0)),
        ],
        out_specs=pl.BlockSpec((blk, D), lambda i: (i, 0)),
        out_shape=jax.ShapeDtypeStruct((NPAD, D), jnp.float32),
    )(x_pad, wt)


def kernel(x, edge_index, conv_w, conv_b, lin_w, lin_b):
    f32 = jnp.float32
    wc = (lin_w @ conv_w) * (1.0 / K)
    bias = lin_w @ conv_b + lin_b
    bias16 = jnp.zeros((D,), f32).at[:7].set(bias)
    wcp = jnp.zeros((D, 128), f32).at[:7, :].set(wc)

    x_pad = jnp.zeros((NPAD, 128), f32).at[:N].set(x)
    z = _tc_project(x_pad, wcp.T)

    row = edge_index[0]
    col = edge_index[1]
    pad = jnp.full((EPAD - E,), N, jnp.int32)
    row_p = jnp.concatenate([row, pad]).reshape(TILES, SCH, CHUNK)
    row_g = jnp.zeros((TILES, RCH, CHUNK), jnp.int32).at[:, :SCH, :].set(row_p)
    col_g = jnp.concatenate([col, pad]).reshape(TILES, SCH, CHUNK)

    h = _sc_propagate(z, row_g, col_g, bias16)
    return h[:N, :7]


# speedup vs baseline: 1.0757x; 1.0042x over previous
import functools

import jax
import jax.numpy as jnp
from jax import lax
from jax.experimental import pallas as pl
from jax.experimental.pallas import tpu as pltpu
from jax.experimental.pallas import tpu_sc as plsc

N = 10000
NPAD = 10240
E = 320000
K = 8
D = 16
TILES = 16
RPT = NPAD // TILES
CHUNK = 256
SCH = 80
RCH = SCH + 2
EPT = SCH * CHUNK
EPAD = EPT * TILES


def _rsqrt16(x):
    i = plsc.bitcast(x, jnp.int32)
    i = 0x5F3759DF - lax.shift_right_logical(i, 1)
    y = plsc.bitcast(i, jnp.float32)
    for _ in range(3):
        y = y * (1.5 - 0.5 * x * y * y)
    return y


def _sc_propagate(z, row_g, col_g, bias16):
    mesh = plsc.VectorSubcoreMesh(
        core_axis_name="c", subcore_axis_name="s", num_cores=1
    )

    @functools.partial(
        pl.kernel,
        out_type=jax.ShapeDtypeStruct((NPAD, D), jnp.float32),
        mesh=mesh,
        scratch_types=[
            pltpu.VMEM_SHARED((NPAD, D), jnp.float32),
            pltpu.VMEM_SHARED((NPAD, D), jnp.float32),
            pltpu.VMEM((RCH, CHUNK), jnp.int32),
            pltpu.VMEM((SCH, CHUNK), jnp.int32),
            pltpu.VMEM((CHUNK, D), jnp.float32),
            pltpu.VMEM((CHUNK, D), jnp.float32),
            pltpu.VMEM((RPT, D), jnp.float32),
            pltpu.VMEM((RPT, D), jnp.float32),
            pltpu.VMEM((RPT, D), jnp.float32),
            pltpu.VMEM((RPT, D), jnp.float32),
            pltpu.VMEM((D,), jnp.float32),
            pltpu.SemaphoreType.DMA,
            pltpu.SemaphoreType.DMA,
        ],
        compiler_params=pltpu.CompilerParams(
            needs_layout_passes=False, use_tc_tiling_on_sc=False
        ),
    )
    def body(z_hbm, row_hbm, col_hbm, bias_hbm, h_hbm,
             nxt_sh, t_sh, row_v, col_v, msg0, msg1, tv, hv, dv, av, bv,
             sem0, sem1):
        tid = lax.axis_index("s")
        sl = pl.ds(tid * RPT, RPT)

        pltpu.sync_copy(row_hbm.at[tid], row_v)
        pltpu.sync_copy(col_hbm.at[tid], col_v)
        pltpu.sync_copy(bias_hbm, bv)

        def fill_ones(i, c):
            msg0[i, :] = jnp.ones((D,), jnp.float32)
            return c

        def fill_zero(i, c):
            av[i, :] = jnp.zeros((D,), jnp.float32)
            return c

        lax.fori_loop(0, CHUNK, fill_ones, 0)
        lax.fori_loop(0, RPT, fill_zero, 0)
        pltpu.sync_copy(av, nxt_sh.at[sl])
        plsc.subcore_barrier()

        def deg_fire(j, c):
            pltpu.async_copy(msg0, nxt_sh.at[col_v.at[j]], ssem_d, add=True)
            return c

        def deg_drain(j, c):
            pltpu.make_async_copy(msg0, nxt_sh.at[pl.ds(0, CHUNK)],
                                  ssem_d).wait()
            return c

        ssem_d = sem1
        lax.fori_loop(0, SCH, deg_fire, 0)
        lax.fori_loop(0, SCH, deg_drain, 0)
        plsc.subcore_barrier()

        pltpu.sync_copy(nxt_sh.at[sl], av)
        pltpu.sync_copy(z_hbm.at[sl], tv)
        b = bv[:]

        def init_rows(i, c):
            deg = av[i, :] + 1.0
            y = _rsqrt16(deg)
            dv[i, :] = y
            tv[i, :] = y * tv[i, :]
            hv[i, :] = b
            av[i, :] = jnp.zeros((D,), jnp.float32)
            return c

        lax.fori_loop(0, RPT, init_rows, 0)
        pltpu.sync_copy(tv, t_sh.at[sl])
        pltpu.sync_copy(av, nxt_sh.at[sl])
        plsc.subcore_barrier()

        def gather(j, buf, sem):
            pltpu.async_copy(t_sh.at[row_v.at[j]], buf, sem)

        def gwait(buf, sem):
            pltpu.make_async_copy(t_sh.at[pl.ds(0, CHUNK)], buf, sem).wait()

        def hop(k, c):
            gather(0, msg0, sem0)
            gather(1, msg1, sem1)

            def pipe(i, cc):
                jj = 2 * i
                gwait(msg0, sem0)
                pltpu.sync_copy(msg0, nxt_sh.at[col_v.at[jj]], add=True)
                gather(jj + 2, msg0, sem0)
                gwait(msg1, sem1)
                pltpu.sync_copy(msg1, nxt_sh.at[col_v.at[jj + 1]], add=True)
                gather(jj + 3, msg1, sem1)
                return cc

            lax.fori_loop(0, SCH // 2, pipe, 0)
            gwait(msg0, sem0)
            gwait(msg1, sem1)
            plsc.subcore_barrier()

            pltpu.sync_copy(nxt_sh.at[sl], av)

            def rowu(i, cc):
                cur = dv[i, :] * (av[i, :] + tv[i, :])
                hv[i, :] = hv[i, :] + cur
                tv[i, :] = dv[i, :] * cur
                av[i, :] = jnp.zeros((D,), jnp.float32)
                return cc

            lax.fori_loop(0, RPT, rowu, 0)
            pltpu.sync_copy(tv, t_sh.at[sl])
            pltpu.sync_copy(av, nxt_sh.at[sl])
            plsc.subcore_barrier()
            return c

        lax.fori_loop(0, K, hop, 0)
        pltpu.sync_copy(hv, h_hbm.at[sl])

    return body(z, row_g, col_g, bias16)


def _tc_project(x_pad, wt):
    def mm(x_ref, w_ref, o_ref):
        o_ref[:, :] = jnp.dot(x_ref[:, :], w_ref[:, :],
                              preferred_element_type=jnp.float32)

    blk = 1024
    return pl.pallas_call(
        mm,
        grid=(NPAD // blk,),
        in_specs=[
            pl.BlockSpec((blk, 128), lambda i: (i, 0)),
            pl.BlockSpec((128, D), lambda i: (0, 0)),
        ],
        out_specs=pl.BlockSpec((blk, D), lambda i: (i, 0)),
        out_shape=jax.ShapeDtypeStruct((NPAD, D), jnp.float32),
    )(x_pad, wt)


def kernel(x, edge_index, conv_w, conv_b, lin_w, lin_b):
    f32 = jnp.float32
    wc = (lin_w @ conv_w) * (1.0 / K)
    bias = lin_w @ conv_b + lin_b
    bias16 = jnp.zeros((D,), f32).at[:7].set(bias)
    wcp = jnp.zeros((D, 128), f32).at[:7, :].set(wc)

    x_pad = jnp.zeros((NPAD, 128), f32).at[:N].set(x)
    z = _tc_project(x_pad, wcp.T)

    row = edge_index[0]
    col = edge_index[1]
    pad = jnp.full((EPAD - E,), N, jnp.int32)
    row_p = jnp.concatenate([row, pad]).reshape(TILES, SCH, CHUNK)
    row_g = jnp.zeros((TILES, RCH, CHUNK), jnp.int32).at[:, :SCH, :].set(row_p)
    col_g = jnp.concatenate([col, pad]).reshape(TILES, SCH, CHUNK)

    h = _sc_propagate(z, row_g, col_g, bias16)
    return h[:N, :7]


# speedup vs baseline: 1.1177x; 1.0391x over previous
import functools

import jax
import jax.numpy as jnp
from jax import lax
from jax.experimental import pallas as pl
from jax.experimental.pallas import tpu as pltpu
from jax.experimental.pallas import tpu_sc as plsc

N = 10000
NPAD = 10240
E = 320000
K = 8
D = 16
TILES = 16
RPT = NPAD // TILES
CHUNK = 128
SCH = 158
RCH = SCH + 2
EPT = SCH * CHUNK
EPAD = EPT * TILES


def _rsqrt16(x):
    i = plsc.bitcast(x, jnp.int32)
    i = 0x5F3759DF - lax.shift_right_logical(i, 1)
    y = plsc.bitcast(i, jnp.float32)
    for _ in range(3):
        y = y * (1.5 - 0.5 * x * y * y)
    return y


def _sc_propagate(z, row_g, col_g, bias16):
    mesh = plsc.VectorSubcoreMesh(
        core_axis_name="c", subcore_axis_name="s", num_cores=1
    )

    @functools.partial(
        pl.kernel,
        out_type=jax.ShapeDtypeStruct((NPAD, D), jnp.float32),
        mesh=mesh,
        scratch_types=[
            pltpu.VMEM_SHARED((NPAD, D), jnp.float32),
            pltpu.VMEM_SHARED((NPAD, D), jnp.float32),
            pltpu.VMEM((RCH, CHUNK), jnp.int32),
            pltpu.VMEM((SCH, CHUNK), jnp.int32),
            pltpu.VMEM((CHUNK, D), jnp.float32),
            pltpu.VMEM((CHUNK, D), jnp.float32),
            pltpu.VMEM((RPT, D), jnp.float32),
            pltpu.VMEM((RPT, D), jnp.float32),
            pltpu.VMEM((RPT, D), jnp.float32),
            pltpu.VMEM((RPT, D), jnp.float32),
            pltpu.VMEM((D,), jnp.float32),
            pltpu.SemaphoreType.DMA,
            pltpu.SemaphoreType.DMA,
        ],
        compiler_params=pltpu.CompilerParams(
            needs_layout_passes=False, use_tc_tiling_on_sc=False
        ),
    )
    def body(z_hbm, row_hbm, col_hbm, bias_hbm, h_hbm,
             nxt_sh, t_sh, row_v, col_v, msg0, msg1, tv, hv, dv, av, bv,
             sem0, sem1):
        tid = lax.axis_index("s")
        sl = pl.ds(tid * RPT, RPT)

        pltpu.sync_copy(row_hbm.at[tid], row_v)
        pltpu.sync_copy(col_hbm.at[tid], col_v)
        pltpu.sync_copy(bias_hbm, bv)

        def fill_ones(i, c):
            msg0[i, :] = jnp.ones((D,), jnp.float32)
            return c

        def fill_zero(i, c):
            av[i, :] = jnp.zeros((D,), jnp.float32)
            return c

        lax.fori_loop(0, CHUNK, fill_ones, 0)
        lax.fori_loop(0, RPT, fill_zero, 0)
        pltpu.sync_copy(av, nxt_sh.at[sl])
        plsc.subcore_barrier()

        def deg_fire(j, c):
            pltpu.async_copy(msg0, nxt_sh.at[col_v.at[j]], ssem_d, add=True)
            return c

        def deg_drain(j, c):
            pltpu.make_async_copy(msg0, nxt_sh.at[pl.ds(0, CHUNK)],
                                  ssem_d).wait()
            return c

        ssem_d = sem1
        lax.fori_loop(0, SCH, deg_fire, 0)
        lax.fori_loop(0, SCH, deg_drain, 0)
        plsc.subcore_barrier()

        pltpu.sync_copy(nxt_sh.at[sl], av)
        pltpu.sync_copy(z_hbm.at[sl], tv)
        b = bv[:]

        def init_rows(i, c):
            deg = av[i, :] + 1.0
            y = _rsqrt16(deg)
            dv[i, :] = y
            tv[i, :] = y * tv[i, :]
            hv[i, :] = b
            av[i, :] = jnp.zeros((D,), jnp.float32)
            return c

        lax.fori_loop(0, RPT, init_rows, 0)
        pltpu.sync_copy(tv, t_sh.at[sl])
        pltpu.sync_copy(av, nxt_sh.at[sl])
        plsc.subcore_barrier()

        def gather(j, buf, sem):
            pltpu.async_copy(t_sh.at[row_v.at[j]], buf, sem)

        def gwait(buf, sem):
            pltpu.make_async_copy(t_sh.at[pl.ds(0, CHUNK)], buf, sem).wait()

        def hop(k, c):
            gather(0, msg0, sem0)
            gather(1, msg1, sem1)

            def pipe(i, cc):
                jj = 2 * i
                gwait(msg0, sem0)
                pltpu.sync_copy(msg0, nxt_sh.at[col_v.at[jj]], add=True)
                gather(jj + 2, msg0, sem0)
                gwait(msg1, sem1)
                pltpu.sync_copy(msg1, nxt_sh.at[col_v.at[jj + 1]], add=True)
                gather(jj + 3, msg1, sem1)
                return cc

            lax.fori_loop(0, SCH // 2, pipe, 0)
            gwait(msg0, sem0)
            gwait(msg1, sem1)
            plsc.subcore_barrier()

            pltpu.sync_copy(nxt_sh.at[sl], av)

            def rowu(i, cc):
                cur = dv[i, :] * (av[i, :] + tv[i, :])
                hv[i, :] = hv[i, :] + cur
                tv[i, :] = dv[i, :] * cur
                av[i, :] = jnp.zeros((D,), jnp.float32)
                return cc

            lax.fori_loop(0, RPT, rowu, 0)
            pltpu.sync_copy(tv, t_sh.at[sl])
            pltpu.sync_copy(av, nxt_sh.at[sl])
            plsc.subcore_barrier()
            return c

        lax.fori_loop(0, K, hop, 0)
        pltpu.sync_copy(hv, h_hbm.at[sl])

    return body(z, row_g, col_g, bias16)


def _tc_project(x_pad, wt):
    def mm(x_ref, w_ref, o_ref):
        o_ref[:, :] = jnp.dot(x_ref[:, :], w_ref[:, :],
                              preferred_element_type=jnp.float32)

    blk = 1024
    return pl.pallas_call(
        mm,
        grid=(NPAD // blk,),
        in_specs=[
            pl.BlockSpec((blk, 128), lambda i: (i, 0)),
            pl.BlockSpec((128, D), lambda i: (0, 0)),
        ],
        out_specs=pl.BlockSpec((blk, D), lambda i: (i, 0)),
        out_shape=jax.ShapeDtypeStruct((NPAD, D), jnp.float32),
    )(x_pad, wt)


def kernel(x, edge_index, conv_w, conv_b, lin_w, lin_b):
    f32 = jnp.float32
    wc = (lin_w @ conv_w) * (1.0 / K)
    bias = lin_w @ conv_b + lin_b
    bias16 = jnp.zeros((D,), f32).at[:7].set(bias)
    wcp = jnp.zeros((D, 128), f32).at[:7, :].set(wc)

    x_pad = jnp.zeros((NPAD, 128), f32).at[:N].set(x)
    z = _tc_project(x_pad, wcp.T)

    row = edge_index[0]
    col = edge_index[1]
    pad = jnp.full((EPAD - E,), N, jnp.int32)
    row_p = jnp.concatenate([row, pad]).reshape(TILES, SCH, CHUNK)
    row_g = jnp.zeros((TILES, RCH, CHUNK), jnp.int32).at[:, :SCH, :].set(row_p)
    col_g = jnp.concatenate([col, pad]).reshape(TILES, SCH, CHUNK)

    h = _sc_propagate(z, row_g, col_g, bias16)
    return h[:N, :7]


# speedup vs baseline: 1.1807x; 1.0563x over previous
import functools

import jax
import jax.numpy as jnp
from jax import lax
from jax.experimental import pallas as pl
from jax.experimental.pallas import tpu as pltpu
from jax.experimental.pallas import tpu_sc as plsc

N = 10000
NPAD = 10240
E = 320000
K = 8
D = 16
TILES = 16
RPT = NPAD // TILES
CHUNK = 128
SCH = 158
RCH = SCH + 2
EPT = SCH * CHUNK
EPAD = EPT * TILES


def _rsqrt16(x):
    i = plsc.bitcast(x, jnp.int32)
    i = 0x5F3759DF - lax.shift_right_logical(i, 1)
    y = plsc.bitcast(i, jnp.float32)
    for _ in range(3):
        y = y * (1.5 - 0.5 * x * y * y)
    return y


def _sc_propagate(z, row_g, col_g, bias16):
    mesh = plsc.VectorSubcoreMesh(
        core_axis_name="c", subcore_axis_name="s", num_cores=1
    )

    @functools.partial(
        pl.kernel,
        out_type=jax.ShapeDtypeStruct((NPAD, D), jnp.float32),
        mesh=mesh,
        scratch_types=[
            pltpu.VMEM_SHARED((NPAD, D), jnp.float32),
            pltpu.VMEM_SHARED((NPAD, D), jnp.float32),
            pltpu.VMEM((RCH, CHUNK), jnp.int32),
            pltpu.VMEM((SCH, CHUNK), jnp.int32),
            pltpu.VMEM((CHUNK, D), jnp.float32),
            pltpu.VMEM((CHUNK, D), jnp.float32),
            pltpu.VMEM((RPT, D), jnp.float32),
            pltpu.VMEM((RPT, D), jnp.float32),
            pltpu.VMEM((RPT, D), jnp.float32),
            pltpu.VMEM((RPT, D), jnp.float32),
            pltpu.VMEM((D,), jnp.float32),
            pltpu.SemaphoreType.DMA,
            pltpu.SemaphoreType.DMA,
        ],
        compiler_params=pltpu.CompilerParams(
            needs_layout_passes=False, use_tc_tiling_on_sc=False
        ),
    )
    def body(z_hbm, row_hbm, col_hbm, bias_hbm, h_hbm,
             nxt_sh, t_sh, row_v, col_v, msg0, msg1, tv, hv, dv, av, bv,
             sem0, sem1):
        tid = lax.axis_index("s")
        sl = pl.ds(tid * RPT, RPT)

        pltpu.sync_copy(row_hbm.at[tid], row_v)
        pltpu.sync_copy(col_hbm.at[tid], col_v)
        pltpu.sync_copy(bias_hbm, bv)

        @plsc.parallel_loop(0, CHUNK, unroll=4)
        def fill_ones(i):
            msg0[i, :] = jnp.ones((D,), jnp.float32)

        @plsc.parallel_loop(0, RPT, unroll=4)
        def fill_zero(i):
            av[i, :] = jnp.zeros((D,), jnp.float32)
        pltpu.sync_copy(av, nxt_sh.at[sl])
        plsc.subcore_barrier()

        def deg_fire(j, c):
            pltpu.async_copy(msg0, nxt_sh.at[col_v.at[j]], ssem_d, add=True)
            return c

        def deg_drain(j, c):
            pltpu.make_async_copy(msg0, nxt_sh.at[pl.ds(0, CHUNK)],
                                  ssem_d).wait()
            return c

        ssem_d = sem1
        lax.fori_loop(0, SCH, deg_fire, 0)
        lax.fori_loop(0, SCH, deg_drain, 0)
        plsc.subcore_barrier()

        pltpu.sync_copy(nxt_sh.at[sl], av)
        pltpu.sync_copy(z_hbm.at[sl], tv)
        b = bv[:]

        @plsc.parallel_loop(0, RPT, unroll=4)
        def init_rows(i):
            deg = av[i, :] + 1.0
            y = _rsqrt16(deg)
            dv[i, :] = y
            tv[i, :] = y * tv[i, :]
            hv[i, :] = b
            av[i, :] = jnp.zeros((D,), jnp.float32)
        pltpu.sync_copy(tv, t_sh.at[sl])
        pltpu.sync_copy(av, nxt_sh.at[sl])
        plsc.subcore_barrier()

        def gather(j, buf, sem):
            pltpu.async_copy(t_sh.at[row_v.at[j]], buf, sem)

        def gwait(buf, sem):
            pltpu.make_async_copy(t_sh.at[pl.ds(0, CHUNK)], buf, sem).wait()

        def hop(k, c):
            gather(0, msg0, sem0)
            gather(1, msg1, sem1)

            def pipe(i, cc):
                jj = 2 * i
                gwait(msg0, sem0)
                pltpu.sync_copy(msg0, nxt_sh.at[col_v.at[jj]], add=True)
                gather(jj + 2, msg0, sem0)
                gwait(msg1, sem1)
                pltpu.sync_copy(msg1, nxt_sh.at[col_v.at[jj + 1]], add=True)
                gather(jj + 3, msg1, sem1)
                return cc

            lax.fori_loop(0, SCH // 2, pipe, 0)
            gwait(msg0, sem0)
            gwait(msg1, sem1)
            plsc.subcore_barrier()

            pltpu.sync_copy(nxt_sh.at[sl], av)

            @plsc.parallel_loop(0, RPT, unroll=4)
            def rowu(i):
                cur = dv[i, :] * (av[i, :] + tv[i, :])
                hv[i, :] = hv[i, :] + cur
                tv[i, :] = dv[i, :] * cur
                av[i, :] = jnp.zeros((D,), jnp.float32)
            pltpu.sync_copy(tv, t_sh.at[sl])
            pltpu.sync_copy(av, nxt_sh.at[sl])
            plsc.subcore_barrier()
            return c

        lax.fori_loop(0, K, hop, 0)
        pltpu.sync_copy(hv, h_hbm.at[sl])

    return body(z, row_g, col_g, bias16)


def _tc_project(x_pad, wt):
    def mm(x_ref, w_ref, o_ref):
        o_ref[:, :] = jnp.dot(x_ref[:, :], w_ref[:, :],
                              preferred_element_type=jnp.float32)

    blk = 1024
    return pl.pallas_call(
        mm,
        grid=(NPAD // blk,),
        in_specs=[
            pl.BlockSpec((blk, 128), lambda i: (i, 0)),
            pl.BlockSpec((128, D), lambda i: (0, 0)),
        ],
        out_specs=pl.BlockSpec((blk, D), lambda i: (i, 0)),
        out_shape=jax.ShapeDtypeStruct((NPAD, D), jnp.float32),
    )(x_pad, wt)


def kernel(x, edge_index, conv_w, conv_b, lin_w, lin_b):
    f32 = jnp.float32
    wc = (lin_w @ conv_w) * (1.0 / K)
    bias = lin_w @ conv_b + lin_b
    bias16 = jnp.zeros((D,), f32).at[:7].set(bias)
    wcp = jnp.zeros((D, 128), f32).at[:7, :].set(wc)

    x_pad = jnp.zeros((NPAD, 128), f32).at[:N].set(x)
    z = _tc_project(x_pad, wcp.T)

    row = edge_index[0]
    col = edge_index[1]
    pad = jnp.full((EPAD - E,), N, jnp.int32)
    row_p = jnp.concatenate([row, pad]).reshape(TILES, SCH, CHUNK)
    row_g = jnp.zeros((TILES, RCH, CHUNK), jnp.int32).at[:, :SCH, :].set(row_p)
    col_g = jnp.concatenate([col, pad]).reshape(TILES, SCH, CHUNK)

    h = _sc_propagate(z, row_g, col_g, bias16)
    return h[:N, :7]


# speedup vs baseline: 1.1821x; 1.0012x over previous
import functools

import jax
import jax.numpy as jnp
from jax import lax
from jax.experimental import pallas as pl
from jax.experimental.pallas import tpu as pltpu
from jax.experimental.pallas import tpu_sc as plsc

N = 10000
NPAD = 10240
E = 320000
K = 8
D = 16
TILES = 16
RPT = NPAD // TILES
CHUNK = 128
SCH = 158
RCH = SCH + 2
EPT = SCH * CHUNK
EPAD = EPT * TILES


def _rsqrt16(x):
    i = plsc.bitcast(x, jnp.int32)
    i = 0x5F3759DF - lax.shift_right_logical(i, 1)
    y = plsc.bitcast(i, jnp.float32)
    for _ in range(3):
        y = y * (1.5 - 0.5 * x * y * y)
    return y


def _sc_propagate(z, row_g, col_g, bias16):
    mesh = plsc.VectorSubcoreMesh(
        core_axis_name="c", subcore_axis_name="s", num_cores=1
    )

    @functools.partial(
        pl.kernel,
        out_type=jax.ShapeDtypeStruct((NPAD, D), jnp.float32),
        mesh=mesh,
        scratch_types=[
            pltpu.VMEM_SHARED((NPAD, D), jnp.float32),
            pltpu.VMEM_SHARED((NPAD, D), jnp.float32),
            pltpu.VMEM((RCH, CHUNK), jnp.int32),
            pltpu.VMEM((SCH, CHUNK), jnp.int32),
            pltpu.VMEM((CHUNK, D), jnp.float32),
            pltpu.VMEM((CHUNK, D), jnp.float32),
            pltpu.VMEM((RPT, D), jnp.float32),
            pltpu.VMEM((RPT, D), jnp.float32),
            pltpu.VMEM((RPT, D), jnp.float32),
            pltpu.VMEM((RPT, D), jnp.float32),
            pltpu.VMEM((D,), jnp.float32),
            pltpu.SemaphoreType.DMA,
            pltpu.SemaphoreType.DMA,
        ],
        compiler_params=pltpu.CompilerParams(
            needs_layout_passes=False, use_tc_tiling_on_sc=False
        ),
    )
    def body(z_hbm, row_hbm, col_hbm, bias_hbm, h_hbm,
             nxt_sh, t_sh, row_v, col_v, msg0, msg1, tv, hv, dv, av, bv,
             sem0, sem1):
        tid = lax.axis_index("s")
        sl = pl.ds(tid * RPT, RPT)

        pltpu.sync_copy(row_hbm.at[tid], row_v)
        pltpu.sync_copy(col_hbm.at[tid], col_v)
        pltpu.sync_copy(bias_hbm, bv)

        @plsc.parallel_loop(0, CHUNK, unroll=8)
        def fill_ones(i):
            msg0[i, :] = jnp.ones((D,), jnp.float32)

        @plsc.parallel_loop(0, RPT, unroll=8)
        def fill_zero(i):
            av[i, :] = jnp.zeros((D,), jnp.float32)
        pltpu.sync_copy(av, nxt_sh.at[sl])
        plsc.subcore_barrier()

        def deg_fire(j, c):
            pltpu.async_copy(msg0, nxt_sh.at[col_v.at[j]], ssem_d, add=True)
            return c

        def deg_drain(j, c):
            pltpu.make_async_copy(msg0, nxt_sh.at[pl.ds(0, CHUNK)],
                                  ssem_d).wait()
            return c

        ssem_d = sem1
        lax.fori_loop(0, SCH, deg_fire, 0)
        lax.fori_loop(0, SCH, deg_drain, 0)
        plsc.subcore_barrier()

        pltpu.sync_copy(nxt_sh.at[sl], av)
        pltpu.sync_copy(z_hbm.at[sl], tv)
        b = bv[:]

        @plsc.parallel_loop(0, RPT, unroll=8)
        def init_rows(i):
            deg = av[i, :] + 1.0
            y = _rsqrt16(deg)
            dv[i, :] = y
            tv[i, :] = y * tv[i, :]
            hv[i, :] = b
            av[i, :] = jnp.zeros((D,), jnp.float32)
        pltpu.sync_copy(tv, t_sh.at[sl])
        pltpu.sync_copy(av, nxt_sh.at[sl])
        plsc.subcore_barrier()

        def gather(j, buf, sem):
            pltpu.async_copy(t_sh.at[row_v.at[j]], buf, sem)

        def gwait(buf, sem):
            pltpu.make_async_copy(t_sh.at[pl.ds(0, CHUNK)], buf, sem).wait()

        def hop(k, c):
            gather(0, msg0, sem0)
            gather(1, msg1, sem1)

            def pipe(i, cc):
                jj = 2 * i
                gwait(msg0, sem0)
                pltpu.sync_copy(msg0, nxt_sh.at[col_v.at[jj]], add=True)
                gather(jj + 2, msg0, sem0)
                gwait(msg1, sem1)
                pltpu.sync_copy(msg1, nxt_sh.at[col_v.at[jj + 1]], add=True)
                gather(jj + 3, msg1, sem1)
                return cc

            lax.fori_loop(0, SCH // 2, pipe, 0)
            gwait(msg0, sem0)
            gwait(msg1, sem1)
            plsc.subcore_barrier()

            pltpu.sync_copy(nxt_sh.at[sl], av)

            @plsc.parallel_loop(0, RPT, unroll=8)
            def rowu(i):
                cur = dv[i, :] * (av[i, :] + tv[i, :])
                hv[i, :] = hv[i, :] + cur
                tv[i, :] = dv[i, :] * cur
                av[i, :] = jnp.zeros((D,), jnp.float32)
            pltpu.sync_copy(tv, t_sh.at[sl])
            pltpu.sync_copy(av, nxt_sh.at[sl])
            plsc.subcore_barrier()
            return c

        lax.fori_loop(0, K, hop, 0)
        pltpu.sync_copy(hv, h_hbm.at[sl])

    return body(z, row_g, col_g, bias16)


def _tc_project(x_pad, wt):
    def mm(x_ref, w_ref, o_ref):
        o_ref[:, :] = jnp.dot(x_ref[:, :], w_ref[:, :],
                              preferred_element_type=jnp.float32)

    blk = 1024
    return pl.pallas_call(
        mm,
        grid=(NPAD // blk,),
        in_specs=[
            pl.BlockSpec((blk, 128), lambda i: (i, 0)),
            pl.BlockSpec((128, D), lambda i: (0, 0)),
        ],
        out_specs=pl.BlockSpec((blk, D), lambda i: (i, 0)),
        out_shape=jax.ShapeDtypeStruct((NPAD, D), jnp.float32),
    )(x_pad, wt)


def kernel(x, edge_index, conv_w, conv_b, lin_w, lin_b):
    f32 = jnp.float32
    wc = (lin_w @ conv_w) * (1.0 / K)
    bias = lin_w @ conv_b + lin_b
    bias16 = jnp.zeros((D,), f32).at[:7].set(bias)
    wcp = jnp.zeros((D, 128), f32).at[:7, :].set(wc)

    x_pad = jnp.zeros((NPAD, 128), f32).at[:N].set(x)
    z = _tc_project(x_pad, wcp.T)

    row = edge_index[0]
    col = edge_index[1]
    pad = jnp.full((EPAD - E,), N, jnp.int32)
    row_p = jnp.concatenate([row, pad]).reshape(TILES, SCH, CHUNK)
    row_g = jnp.zeros((TILES, RCH, CHUNK), jnp.int32).at[:, :SCH, :].set(row_p)
    col_g = jnp.concatenate([col, pad]).reshape(TILES, SCH, CHUNK)

    h = _sc_propagate(z, row_g, col_g, bias16)
    return h[:N, :7]
